# Initial kernel scaffold; baseline (speedup 1.0000x reference)
import jax
import jax.numpy as jnp
from jax.experimental import pallas as pl


def kernel(x, edge_index, batch, y, params):
    raise NotImplementedError("write your pallas kernel here")


# speedup vs baseline: 5.2335x; 5.2335x over previous
import functools

import jax
import jax.numpy as jnp
from jax import lax
from jax.experimental import pallas as pl
from jax.experimental.pallas import tpu as pltpu
from jax.experimental.pallas import tpu_sc as plsc

N_NODES = 10000
N_PAD = 10240
N_EDGES = 320000
H = 128
B_GR = 64
NUM_LAYERS = 7

NC = 2
NS = 16
LANES = 16
EDGES_PER_TILE = N_EDGES // (NC * NS)
CHUNK = 80
N_CHUNKS = EDGES_PER_TILE // CHUNK
ROWS_PER_TILE = N_PAD // NS

BLK = 1280
GRID = N_PAD // BLK

f32 = jnp.float32


def _silu(v):
    return v * (1.0 / (1.0 + jnp.exp(-v)))


def _make_edge_kernel():
    CH = 80
    STAGE = 2000
    mesh = plsc.VectorSubcoreMesh(core_axis_name="c", subcore_axis_name="s",
                                  num_cores=NC, num_subcores=NS)

    out_type = [jax.ShapeDtypeStruct((NC * N_PAD, H), f32)]
    scratch = [
        pltpu.VMEM((STAGE,), jnp.int32),
        pltpu.VMEM((STAGE,), jnp.int32),
        pltpu.VMEM((CH, H), f32),
        pltpu.VMEM((CH, H), f32),
        pltpu.VMEM((CH, H), f32),
        pltpu.VMEM_SHARED((N_PAD, H), f32),
        pltpu.SemaphoreType.DMA,
        pltpu.SemaphoreType.DMA,
    ]

    def body(a_hbm, b_hbm, src_hbm, dst_hbm, s_out,
             idx_s, idx_d, a_buf, b_buf, out_buf, s_sp, sem1, sem2):
        cid = lax.axis_index("c")
        sid = lax.axis_index("s")

        zero = jnp.zeros((LANES,), f32)

        def zero_row(r, carry):
            for g in range(H // LANES):
                out_buf[r, pl.ds(g * LANES, LANES)] = zero
            return carry

        lax.fori_loop(0, CH, zero_row, 0)
        for k in range(ROWS_PER_TILE // CH):
            pltpu.sync_copy(
                out_buf, s_sp.at[pl.ds(sid * ROWS_PER_TILE + k * CH, CH)])
        plsc.subcore_barrier()

        ebase = (cid * NS + sid) * EDGES_PER_TILE
        for st in range(EDGES_PER_TILE // STAGE):
            sbase = ebase + st * STAGE
            pltpu.sync_copy(src_hbm.at[pl.ds(sbase, STAGE)], idx_s)
            pltpu.sync_copy(dst_hbm.at[pl.ds(sbase, STAGE)], idx_d)

            @pl.loop(0, STAGE // CH)
            def chunk_body(j):
                isl = pl.ds(j * CH, CH)
                cp1 = pltpu.async_copy(a_hbm.at[idx_s.at[isl]], a_buf, sem1)
                cp2 = pltpu.async_copy(b_hbm.at[idx_d.at[isl]], b_buf, sem2)
                cp1.wait()
                cp2.wait()

                def row(r, c2):
                    for g in range(H // LANES):
                        sl = pl.ds(g * LANES, LANES)
                        t = a_buf[r, sl] + b_buf[r, sl]
                        out_buf[r, sl] = t * (1.0 / (1.0 + jnp.exp(-t)))
                    return c2

                lax.fori_loop(0, CH, row, 0)
                pltpu.sync_copy(out_buf, s_sp.at[idx_d.at[isl]], add=True)

        plsc.subcore_barrier()
        rows = pl.ds(sid * ROWS_PER_TILE, ROWS_PER_TILE)
        orows = pl.ds(cid * N_PAD + sid * ROWS_PER_TILE, ROWS_PER_TILE)
        pltpu.sync_copy(s_sp.at[rows], s_out.at[orows])

    return pl.kernel(body, out_type=out_type, mesh=mesh, scratch_types=scratch)


def _make_deg_kernel():
    CH = 80
    STAGE = 2000
    mesh = plsc.VectorSubcoreMesh(core_axis_name="c", subcore_axis_name="s",
                                  num_cores=NC, num_subcores=NS)

    out_type = [jax.ShapeDtypeStruct((NC * N_PAD, H), f32)]
    scratch = [
        pltpu.VMEM((STAGE,), jnp.int32),
        pltpu.VMEM((CH, H), f32),
        pltpu.VMEM((CH, H), f32),
        pltpu.VMEM_SHARED((N_PAD, H), f32),
    ]

    def body(dst_hbm, d_out, idx_d, zeros_buf, ones_buf, d_sp):
        cid = lax.axis_index("c")
        sid = lax.axis_index("s")

        zero = jnp.zeros((LANES,), f32)
        one = jnp.full((LANES,), 1.0, f32)

        def fill_row(r, carry):
            for g in range(H // LANES):
                sl = pl.ds(g * LANES, LANES)
                zeros_buf[r, sl] = zero
                ones_buf[r, sl] = one
            return carry

        lax.fori_loop(0, CH, fill_row, 0)
        for k in range(ROWS_PER_TILE // CH):
            pltpu.sync_copy(
                zeros_buf, d_sp.at[pl.ds(sid * ROWS_PER_TILE + k * CH, CH)])
        plsc.subcore_barrier()

        ebase = (cid * NS + sid) * EDGES_PER_TILE
        for st in range(EDGES_PER_TILE // STAGE):
            pltpu.sync_copy(dst_hbm.at[pl.ds(ebase + st * STAGE, STAGE)],
                            idx_d)

            @pl.loop(0, STAGE // CH)
            def chunk_body(j):
                pltpu.sync_copy(ones_buf, d_sp.at[idx_d.at[pl.ds(j * CH, CH)]],
                                add=True)

        plsc.subcore_barrier()
        rows = pl.ds(sid * ROWS_PER_TILE, ROWS_PER_TILE)
        orows = pl.ds(cid * N_PAD + sid * ROWS_PER_TILE, ROWS_PER_TILE)
        pltpu.sync_copy(d_sp.at[rows], d_out.at[orows])

    return pl.kernel(body, out_type=out_type, mesh=mesh, scratch_types=scratch)


@functools.lru_cache(maxsize=None)
def _edge_kernel():
    return _make_edge_kernel()


@functools.lru_cache(maxsize=None)
def _deg_kernel():
    return _make_deg_kernel()


def _edge_call(a, b, src, dst):
    (s,) = _edge_kernel()(a, b, src, dst)
    return s.reshape(NC, N_PAD, H)


def _deg_call(dst):
    (d,) = _deg_kernel()(dst)
    d = d.reshape(NC, N_PAD, H)
    return d[0, :, 0:1] + d[1, :, 0:1]


def _dot(a, b):
    return jnp.dot(a, b, preferred_element_type=f32)


def _pre_body(x_ref, ew1, eb1, ew2, eb2, aw, bw, bb, h_ref, a_ref, b_ref):
    h1 = _silu(_dot(x_ref[...], ew1[...]) + eb1[...])
    h = _dot(h1, ew2[...]) + eb2[...]
    h_ref[...] = h
    a_ref[...] = _dot(h, aw[...])
    b_ref[...] = _dot(h, bw[...]) + bb[...]


def _row_spec(w):
    return pl.BlockSpec((BLK, w), lambda i: (i, 0))


def _full_spec(shape):
    return pl.BlockSpec(shape, lambda i: tuple(0 for _ in shape))


def _pre_call(x_pad, ew1, eb1, ew2, eb2, aw, bw, bb):
    return pl.pallas_call(
        _pre_body,
        grid=(GRID,),
        in_specs=[
            _row_spec(8),
            _full_spec((8, H)), _full_spec((1, H)),
            _full_spec((H, H)), _full_spec((1, H)),
            _full_spec((H, H)), _full_spec((H, H)), _full_spec((1, H)),
        ],
        out_specs=[_row_spec(H), _row_spec(H), _row_spec(H)],
        out_shape=[jax.ShapeDtypeStruct((N_PAD, H), f32)] * 3,
    )(x_pad, ew1, eb1, ew2, eb2, aw, bw, bb)


def _make_mid_body(with_next):
    def body(*refs):
        if with_next:
            (h_ref, s_ref, dcol, mw2, mb2, u1a, u1b, ub1, u2, ub2,
             aw, bw, bb, hn_ref, a_ref, b_ref) = refs
        else:
            (h_ref, s_ref, dcol, mw2, mb2, u1a, u1b, ub1, u2, ub2,
             hn_ref) = refs
        sv = s_ref[...]
        s = sv[0, :, 0:H] + sv[1, :, 0:H]
        agg = _dot(s, mw2[...]) + dcol[...] * mb2[...]
        t = _silu(_dot(h_ref[...], u1a[...]) + _dot(agg, u1b[...]) + ub1[...])
        hn = _dot(t, u2[...]) + ub2[...]
        hn_ref[...] = hn
        if with_next:
            a_ref[...] = _dot(hn, aw[...])
            b_ref[...] = _dot(hn, bw[...]) + bb[...]
    return body


def _mid_call(h, s, dcol, mw2, mb2, u1a, u1b, ub1, u2, ub2, nxt):
    with_next = nxt is not None
    sw = s.shape[-1]
    in_specs = [
        _row_spec(H),
        pl.BlockSpec((2, BLK, sw), lambda i: (0, i, 0)),
        _row_spec(1),
        _full_spec((H, H)), _full_spec((1, H)),
        _full_spec((H, H)), _full_spec((H, H)), _full_spec((1, H)),
        _full_spec((H, H)), _full_spec((1, H)),
    ]
    args = [h, s, dcol, mw2, mb2, u1a, u1b, ub1, u2, ub2]
    n_out = 1
    if with_next:
        in_specs += [_full_spec((H, H)), _full_spec((H, H)), _full_spec((1, H))]
        args += list(nxt)
        n_out = 3
    out = pl.pallas_call(
        _make_mid_body(with_next),
        grid=(GRID,),
        in_specs=in_specs,
        out_specs=[_row_spec(H)] * n_out,
        out_shape=[jax.ShapeDtypeStruct((N_PAD, H), f32)] * n_out,
    )(*args)
    return out


def _pool_body(h_ref, bat_ref, y_ref, hw1, hb1, hw2, hb2,
               pooled_ref, counts_ref, bp_ref, loss_ref, acc_ref):
    i = pl.program_id(0)
    oh = (bat_ref[...] == lax.broadcasted_iota(jnp.int32, (BLK, B_GR), 1)
          ).astype(f32)
    pc = lax.dot_general(oh, h_ref[...], (((0,), (0,)), ((), ())),
                         preferred_element_type=f32)
    cc = lax.dot_general(oh, jnp.ones((BLK, H), f32), (((0,), (0,)), ((), ())),
                         preferred_element_type=f32)

    @pl.when(i == 0)
    def _init():
        pooled_ref[...] = pc
        counts_ref[...] = cc

    @pl.when(i > 0)
    def _acc():
        pooled_ref[...] += pc
        counts_ref[...] += cc

    @pl.when(i == pl.num_programs(0) - 1)
    def _final():
        pooled = pooled_ref[...] / jnp.maximum(counts_ref[...], 1.0)
        t = _silu(_dot(pooled, hw1[...]) + hb1[...])
        preds = _dot(t, hw2[...]) + hb2[...]
        colmask = lax.broadcasted_iota(jnp.int32, (B_GR, H), 1) < 2
        y = y_ref[...]
        m = jnp.max(jnp.where(colmask, preds, -jnp.inf), axis=1, keepdims=True)
        se = jnp.sum(jnp.where(colmask, jnp.exp(preds - m), 0.0), axis=1,
                     keepdims=True)
        lse = m + jnp.log(se)
        loss = -jnp.sum(jnp.where(colmask, y * (preds - lse), 0.0), axis=1,
                        keepdims=True)
        loss_ref[...] = loss
        bp_ref[...] = jnp.sum(loss, axis=0, keepdims=True) / B_GR
        ap = preds[:, 1:2] > preds[:, 0:1]
        ay = y[:, 1:2] > y[:, 0:1]
        acc_ref[...] = (ap == ay).astype(f32)


def _pool_call(h, bat_pad, y_pad, hw1, hb1, hw2, hb2):
    outs = pl.pallas_call(
        _pool_body,
        grid=(GRID,),
        in_specs=[
            _row_spec(H),
            _row_spec(1),
            _full_spec((B_GR, H)),
            _full_spec((H, H)), _full_spec((1, H)),
            _full_spec((H, H)), _full_spec((1, H)),
        ],
        out_specs=[
            _full_spec((B_GR, H)), _full_spec((B_GR, H)),
            _full_spec((1, 1)), _full_spec((B_GR, 1)), _full_spec((B_GR, 1)),
        ],
        out_shape=[
            jax.ShapeDtypeStruct((B_GR, H), f32),
            jax.ShapeDtypeStruct((B_GR, H), f32),
            jax.ShapeDtypeStruct((1, 1), f32),
            jax.ShapeDtypeStruct((B_GR, 1), f32),
            jax.ShapeDtypeStruct((B_GR, 1), f32),
        ],
    )(h, bat_pad, y_pad, hw1, hb1, hw2, hb2)
    _, _, bp, loss, acc = outs
    return bp, loss, acc


def kernel(x, edge_index, batch, y, params):
    p = params
    x_pad = jnp.zeros((N_PAD, 8), f32).at[:N_NODES, :3].set(x)
    src = edge_index[0].astype(jnp.int32)
    dst = edge_index[1].astype(jnp.int32)
    bat_pad = jnp.full((N_PAD, 1), B_GR, jnp.int32).at[:N_NODES, 0].set(batch)
    y_pad = jnp.zeros((B_GR, H), f32).at[:, :2].set(y)

    ew1 = jnp.zeros((8, H), f32).at[:3].set(p['emb_W1'])
    eb1 = p['emb_b1'].reshape(1, H)
    ew2 = p['emb_W2']
    eb2 = p['emb_b2'].reshape(1, H)

    def msg_split(lp):
        w1 = lp['msg_W1']
        return w1[:H], w1[H:], lp['msg_b1'].reshape(1, H)

    aw0, bw0, bb0 = msg_split(p['layers'][0])
    h, a, b = _pre_call(x_pad, ew1, eb1, ew2, eb2, aw0, bw0, bb0)
    dcol = _deg_call(dst)
    s = _edge_call(a, b, src, dst)

    for i in range(NUM_LAYERS):
        lp = p['layers'][i]
        mw2 = lp['msg_W2']
        mb2 = lp['msg_b2'].reshape(1, H)
        u1a = lp['upd_W1'][:H]
        u1b = lp['upd_W1'][H:]
        ub1 = lp['upd_b1'].reshape(1, H)
        u2 = lp['upd_W2']
        ub2 = lp['upd_b2'].reshape(1, H)
        if i < NUM_LAYERS - 1:
            nxt = msg_split(p['layers'][i + 1])
            h, a, b = _mid_call(h, s, dcol, mw2, mb2, u1a, u1b, ub1, u2, ub2,
                                nxt)
            s = _edge_call(a, b, src, dst)
        else:
            (h,) = _mid_call(h, s, dcol, mw2, mb2, u1a, u1b, ub1, u2, ub2,
                             None)

    hw1 = p['head_W1']
    hb1 = p['head_b1'].reshape(1, H)
    hw2 = jnp.zeros((H, H), f32).at[:, :2].set(p['head_W2'])
    hb2 = jnp.zeros((1, H), f32).at[0, :2].set(p['head_b2'])

    bp, loss, acc = _pool_call(h, bat_pad, y_pad, hw1, hb1, hw2, hb2)
    return (bp.reshape(()), loss.reshape(B_GR), acc.reshape(B_GR))
